# baseline (speedup 1.0000x reference)
import functools

import jax
import jax.numpy as jnp
from jax.experimental import pallas as pl
from jax.experimental.pallas import tpu as pltpu

EMB = 2048
NE = 16
TOKENS = 4 * 4096
BLK = 1024
NBLK = TOKENS // BLK
NBUF = 4


def _gating_body(x_hbm, wt_ref, gw_hbm, tkw_hbm, tki_hbm,
                 xbuf, gws, tkws, tkis, insems, osems):
    i = pl.program_id(0)

    @pl.when(i == 0)
    def _prolog():
        for b in range(NBUF - 1):
            pltpu.make_async_copy(
                x_hbm.at[pl.ds(b * BLK, BLK), :], xbuf.at[b], insems.at[b]
            ).start()

    nxt = i + NBUF - 1

    @pl.when(nxt < NBLK)
    def _prefetch():
        slot = jax.lax.rem(nxt, NBUF)
        pltpu.make_async_copy(
            x_hbm.at[pl.ds(nxt * BLK, BLK), :], xbuf.at[slot], insems.at[slot]
        ).start()

    cur = jax.lax.rem(i, NBUF)
    pltpu.make_async_copy(
        x_hbm.at[pl.ds(i * BLK, BLK), :], xbuf.at[cur], insems.at[cur]
    ).wait()

    x = xbuf[cur]
    wt = wt_ref[...]
    logits = jnp.dot(x, wt, preferred_element_type=jnp.float32)

    m = jnp.max(logits, axis=-1, keepdims=True)
    e = jnp.exp(logits - m)
    s = jnp.sum(e, axis=-1, keepdims=True)
    gw = e / s

    lane = jax.lax.broadcasted_iota(jnp.int32, gw.shape, 1)
    m1 = jnp.max(gw, axis=-1, keepdims=True)
    i1 = jnp.min(jnp.where(gw == m1, lane, NE), axis=-1, keepdims=True)
    masked = jnp.where(lane == i1, -jnp.inf, gw)
    m2 = jnp.max(masked, axis=-1, keepdims=True)
    i2 = jnp.min(jnp.where(masked == m2, lane, NE), axis=-1, keepdims=True)

    e2 = jnp.exp(m2 - m1)
    denom = 1.0 + e2
    lane2 = jax.lax.broadcasted_iota(jnp.int32, (gw.shape[0], 2), 1)
    tkw = jnp.where(lane2 == 0, 1.0 / denom, e2 / denom)
    tki = jnp.where(lane2 == 0, i1, i2)

    oslot = jax.lax.rem(i, 2)

    def _owaits(slot):
        pltpu.make_async_copy(gws.at[slot], gw_hbm.at[pl.ds(0, BLK), :],
                              osems.at[slot, 0]).wait()
        pltpu.make_async_copy(tkws.at[slot], tkw_hbm.at[pl.ds(0, BLK), :],
                              osems.at[slot, 1]).wait()
        pltpu.make_async_copy(tkis.at[slot], tki_hbm.at[pl.ds(0, BLK), :],
                              osems.at[slot, 2]).wait()

    @pl.when(i >= 2)
    def _wait_prev():
        _owaits(oslot)

    gws[oslot] = gw
    tkws[oslot] = tkw
    tkis[oslot] = tki

    pltpu.make_async_copy(gws.at[oslot], gw_hbm.at[pl.ds(i * BLK, BLK), :],
                          osems.at[oslot, 0]).start()
    pltpu.make_async_copy(tkws.at[oslot], tkw_hbm.at[pl.ds(i * BLK, BLK), :],
                          osems.at[oslot, 1]).start()
    pltpu.make_async_copy(tkis.at[oslot], tki_hbm.at[pl.ds(i * BLK, BLK), :],
                          osems.at[oslot, 2]).start()

    @pl.when(i == NBLK - 1)
    def _drain():
        _owaits(1 - oslot)
        _owaits(oslot)


@functools.partial(jax.jit, static_argnames=("interpret",))
def kernel(x, W, interpret=False):
    xf = x.reshape(TOKENS, EMB)
    wt = W.T
    gw, tkw, tki = pl.pallas_call(
        _gating_body,
        grid=(NBLK,),
        in_specs=[
            pl.BlockSpec(memory_space=pltpu.MemorySpace.HBM),
            pl.BlockSpec((EMB, NE), lambda i: (0, 0)),
        ],
        out_specs=[
            pl.BlockSpec(memory_space=pltpu.MemorySpace.HBM),
            pl.BlockSpec(memory_space=pltpu.MemorySpace.HBM),
            pl.BlockSpec(memory_space=pltpu.MemorySpace.HBM),
        ],
        out_shape=[
            jax.ShapeDtypeStruct((TOKENS, NE), jnp.float32),
            jax.ShapeDtypeStruct((TOKENS, 2), jnp.float32),
            jax.ShapeDtypeStruct((TOKENS, 2), jnp.int32),
        ],
        scratch_shapes=[
            pltpu.MemorySpace.VMEM((NBUF, BLK, EMB), jnp.float32),
            pltpu.MemorySpace.VMEM((2, BLK, NE), jnp.float32),
            pltpu.MemorySpace.VMEM((2, BLK, 2), jnp.float32),
            pltpu.MemorySpace.VMEM((2, BLK, 2), jnp.int32),
            pltpu.SemaphoreType.DMA((NBUF,)),
            pltpu.SemaphoreType.DMA((2, 3)),
        ],
        interpret=interpret,
        compiler_params=pltpu.CompilerParams(
            dimension_semantics=("arbitrary",),
        ),
    )(xf, wt)
    B, S = x.shape[0], x.shape[1]
    return (gw.reshape(B, S, NE), tkw.reshape(B, S, 2), tki.reshape(B, S, 2))


# speedup vs baseline: 1.4725x; 1.4725x over previous
import functools

import jax
import jax.numpy as jnp
from jax.experimental import pallas as pl
from jax.experimental.pallas import tpu as pltpu

EMB = 2048
NE = 16
TOKENS = 4 * 4096
BLK = 1024


def _gating_body(x_ref, wt_ref, gwt_ref, tkwt_ref, tkit_ref):
    x = x_ref[...]
    wt = wt_ref[...]
    logits = jnp.dot(x, wt, preferred_element_type=jnp.float32)
    lg = logits.T

    m = jnp.max(lg, axis=0, keepdims=True)
    e = jnp.exp(lg - m)
    s = jnp.sum(e, axis=0, keepdims=True)
    gw = e / s
    gwt_ref[...] = gw

    row = jax.lax.broadcasted_iota(jnp.int32, gw.shape, 0)
    m1 = jnp.max(gw, axis=0, keepdims=True)
    i1 = jnp.min(jnp.where(gw == m1, row, NE), axis=0, keepdims=True)
    masked = jnp.where(row == i1, -jnp.inf, gw)
    m2 = jnp.max(masked, axis=0, keepdims=True)
    i2 = jnp.min(jnp.where(masked == m2, row, NE), axis=0, keepdims=True)

    e2 = jnp.exp(m2 - m1)
    denom = 1.0 + e2
    row2 = jax.lax.broadcasted_iota(jnp.int32, (2, gw.shape[1]), 0)
    tkwt_ref[...] = jnp.where(row2 == 0, 1.0 / denom, e2 / denom)
    tkit_ref[...] = jnp.where(row2 == 0, i1, i2)


@functools.partial(jax.jit, static_argnames=("interpret",))
def kernel(x, W, interpret=False):
    xf = x.reshape(TOKENS, EMB)
    wt = W.T
    grid = (TOKENS // BLK,)
    gwt, tkwt, tkit = pl.pallas_call(
        _gating_body,
        grid=grid,
        in_specs=[
            pl.BlockSpec((BLK, EMB), lambda i: (i, 0)),
            pl.BlockSpec((EMB, NE), lambda i: (0, 0)),
        ],
        out_specs=[
            pl.BlockSpec((NE, BLK), lambda i: (0, i)),
            pl.BlockSpec((2, BLK), lambda i: (0, i)),
            pl.BlockSpec((2, BLK), lambda i: (0, i)),
        ],
        out_shape=[
            jax.ShapeDtypeStruct((NE, TOKENS), jnp.float32),
            jax.ShapeDtypeStruct((2, TOKENS), jnp.float32),
            jax.ShapeDtypeStruct((2, TOKENS), jnp.int32),
        ],
        interpret=interpret,
        compiler_params=pltpu.CompilerParams(
            dimension_semantics=("arbitrary",),
        ),
    )(xf, wt)
    B, S = x.shape[0], x.shape[1]
    return (gwt.T.reshape(B, S, NE), tkwt.T.reshape(B, S, 2),
            tkit.T.reshape(B, S, 2))
